# baseline (speedup 1.0000x reference)
import functools

import jax
import jax.numpy as jnp
from jax import lax
from jax.experimental import pallas as pl
from jax.experimental.pallas import tpu as pltpu
from jax.experimental.pallas import tpu_sc as plsc

VOCAB = 1000000
EMBED = 64
SEQ = 200
BATCH = 1024

NUM_CORES = 2
NUM_SUBCORES = 16
E_PER_CORE = EMBED // NUM_CORES
SMAX = 13
LANES = 16
CHUNK = 128
NCH = BATCH // CHUNK
VMAIN = 999936


def _sc_detile(table_t, tail_flat):
    mesh = plsc.VectorSubcoreMesh(core_axis_name="c", subcore_axis_name="s")

    @functools.partial(
        pl.kernel,
        mesh=mesh,
        out_type=jax.ShapeDtypeStruct((EMBED * VOCAB,), jnp.float32),
        scratch_types=[
            pltpu.VMEM((47616,), jnp.float32),
            pltpu.VMEM((47616,), jnp.float32),
            pltpu.SemaphoreType.DMA,
            pltpu.SemaphoreType.DMA,
        ],
        compiler_params=pltpu.CompilerParams(use_tc_tiling_on_sc=True,
                                             needs_layout_passes=False),
    )
    def k(tab_hbm, tail_hbm, out_hbm, buf0, buf1, sem_r, sem_w):
        chunks = [(i * 47616, 47616) for i in range(21)]
        _detile_body(tab_hbm, tail_hbm, out_hbm, (buf0, buf1), sem_r,
                     sem_w, chunks)

    return k(table_t, tail_flat)


def _detile_body(tab_hbm, tail_hbm, out_hbm, buf, sem_r, sem_w, chunks):
    c = lax.axis_index("c")
    t = lax.axis_index("s")
    wid = t * NUM_CORES + c
    e0 = wid * 2
    seq = [(e0 + r, off, ln) for r in range(2) for (off, ln) in chunks]
    n = len(seq)
    for r in range(2):
        pltpu.sync_copy(tail_hbm.at[pl.ds((e0 + r) * 64, 64)],
                        buf[0].at[pl.ds(r * 64, 64)])
    for r in range(2):
        pltpu.async_copy(buf[0].at[pl.ds(r * 64, 64)],
                         out_hbm.at[pl.ds((e0 + r) * VOCAB + VMAIN, 64)],
                         sem_w)
    for r in range(2):
        pltpu.make_async_copy(buf[0].at[pl.ds(r * 64, 64)],
                              out_hbm.at[pl.ds((e0 + r) * VOCAB + VMAIN, 64)],
                              sem_w).wait()
    e, off, ln = seq[0]
    pltpu.async_copy(tab_hbm.at[e, pl.ds(off, ln)], buf[0].at[pl.ds(0, ln)],
                     sem_r)
    for i, (e, off, ln) in enumerate(seq):
        b = buf[i % 2]
        pltpu.make_async_copy(tab_hbm.at[e, pl.ds(off, ln)],
                              b.at[pl.ds(0, ln)], sem_r).wait()
        if i >= 1:
            pe, poff, pln = seq[i - 1]
            pltpu.make_async_copy(tab_hbm.at[pe, pl.ds(poff, pln)],
                                  buf[1 - i % 2].at[pl.ds(0, pln)],
                                  sem_w).wait()
        if i + 1 < n:
            ne, noff, nln = seq[i + 1]
            pltpu.async_copy(tab_hbm.at[ne, pl.ds(noff, nln)],
                             buf[1 - i % 2].at[pl.ds(0, nln)], sem_r)
        pltpu.async_copy(b.at[pl.ds(0, ln)],
                         out_hbm.at[pl.ds(e * VOCAB + off, ln)], sem_w)
    e, off, ln = seq[n - 1]
    pltpu.make_async_copy(tab_hbm.at[e, pl.ds(off, ln)],
                          buf[(n - 1) % 2].at[pl.ds(0, ln)], sem_w).wait()


def _sc_embed(idx_flat, table_lin, pos2d):
    mesh = plsc.VectorSubcoreMesh(core_axis_name="c", subcore_axis_name="s")

    @functools.partial(
        pl.kernel,
        mesh=mesh,
        out_type=jax.ShapeDtypeStruct((SEQ, 8, NCH, 8, CHUNK), jnp.float32),
        scratch_types=[
            pltpu.VMEM_SHARED((2, VOCAB), jnp.float32),
            pltpu.VMEM((2, BATCH), jnp.int32),
            pltpu.VMEM((SMAX, EMBED), jnp.float32),
            pltpu.VMEM((2, NCH, CHUNK), jnp.float32),
            pltpu.SemaphoreType.DMA,
            pltpu.SemaphoreType.DMA,
            pltpu.SemaphoreType.DMA,
            pltpu.SemaphoreType.DMA,
        ],
        compiler_params=pltpu.CompilerParams(use_tc_tiling_on_sc=False,
                                             needs_layout_passes=False),
    )
    def k(idx_hbm, tab_hbm, pos_hbm, out_hbm, sp, idx_v, pos_v,
          dst_v, sem_st, sem_ix, sem_g, sem_o):
        c = lax.axis_index("c")
        t = lax.axis_index("s")
        s0 = (t * 25) // 2
        s1 = ((t + 1) * 25) // 2
        ns = s1 - s0
        e_base = c * E_PER_CORE

        pltpu.sync_copy(pos_hbm.at[pl.ds(s0, SMAX)], pos_v)

        @pl.when(t == 0)
        def _():
            pltpu.async_copy(tab_hbm.at[pl.ds(e_base * VOCAB, VOCAB)],
                             sp.at[0], sem_st)

        def e_body(j, carry):
            buf = j % 2
            pltpu.async_copy(idx_hbm.at[pl.ds(s0 * BATCH, BATCH)],
                             idx_v.at[0], sem_ix)
            pltpu.async_copy(idx_hbm.at[pl.ds((s0 + 1) * BATCH, BATCH)],
                             idx_v.at[1], sem_ix)
            plsc.subcore_barrier()

            @pl.when(t == 0)
            def _():
                pltpu.make_async_copy(
                    tab_hbm.at[pl.ds(0, VOCAB)],
                    sp.at[buf], sem_st).wait()

                @pl.when(j + 1 < E_PER_CORE)
                def _():
                    pltpu.async_copy(
                        tab_hbm.at[pl.ds((e_base + j + 1) * VOCAB, VOCAB)],
                        sp.at[(j + 1) % 2], sem_st)

            plsc.subcore_barrier()
            e = e_base + j
            tr = e // 8
            ei = e % 8

            pltpu.make_async_copy(idx_hbm.at[pl.ds(0, BATCH)], idx_v.at[0],
                                  sem_ix).wait()
            for cc in range(NCH):
                pltpu.async_copy(
                    sp.at[buf].at[idx_v.at[0, pl.ds(cc * CHUNK, CHUNK)]],
                    dst_v.at[0, cc], sem_g)

            def s_body(si, carry2):
                p = si % 2
                ps = jnp.zeros((LANES,), jnp.int32) + si
                pe = jnp.zeros((LANES,), jnp.int32) + e
                pos_splat = plsc.load_gather(pos_v, [ps, pe])
                for cc in range(NCH):
                    pltpu.make_async_copy(out_hbm.at[0, 0, 0, 0],
                                          dst_v.at[p, cc], sem_g).wait()
                    for u in range(CHUNK // LANES):
                        plsc.addupdate(
                            dst_v.at[p, cc, pl.ds(u * LANES, LANES)],
                            pos_splat,
                        )

                @pl.when(si + 2 < ns)
                def _():
                    pltpu.async_copy(
                        idx_hbm.at[pl.ds((s0 + si + 2) * BATCH, BATCH)],
                        idx_v.at[p], sem_ix)

                @pl.when(si >= 1)
                def _():
                    pltpu.make_async_copy(out_hbm.at[0, 0, :, 0],
                                          dst_v.at[1 - p], sem_o).wait()

                @pl.when(si + 1 < ns)
                def _():
                    pltpu.make_async_copy(idx_hbm.at[pl.ds(0, BATCH)],
                                          idx_v.at[1 - p], sem_ix).wait()

                    @pl.when((si + 1) % 3 == 2)
                    def _():
                        src = tab_hbm.at[pl.ds(e * VOCAB, VOCAB)]
                        for cc in range(NCH):
                            pltpu.async_copy(
                                src.at[idx_v.at[1 - p,
                                                pl.ds(cc * CHUNK, CHUNK)]],
                                dst_v.at[1 - p, cc], sem_g)

                    @pl.when((si + 1) % 3 != 2)
                    def _():
                        for cc in range(NCH):
                            pltpu.async_copy(
                                sp.at[buf].at[idx_v.at[1 - p,
                                                       pl.ds(cc * CHUNK,
                                                             CHUNK)]],
                                dst_v.at[1 - p, cc], sem_g)

                pltpu.async_copy(dst_v.at[p], out_hbm.at[s0 + si, tr, :, ei],
                                 sem_o)
                return carry2

            lax.fori_loop(0, ns, s_body, 0)
            pltpu.make_async_copy(out_hbm.at[0, 0, :, 0],
                                  dst_v.at[(ns + 1) % 2], sem_o).wait()
            return carry

        lax.fori_loop(0, E_PER_CORE, e_body, 0)

    return k(idx_flat, table_lin, pos2d)


def kernel(idx, token_table, pos_table):
    idx_flat = idx.astype(jnp.int32).T.reshape(SEQ * BATCH)
    tail_flat = token_table[VMAIN:].T.reshape(64 * 64)
    table_lin = _sc_detile(token_table.T, tail_flat)
    out5 = _sc_embed(idx_flat, table_lin, pos_table)
    return out5.transpose(2, 4, 0, 1, 3).reshape(BATCH, SEQ, EMBED)


# speedup vs baseline: 1.3800x; 1.3800x over previous
import functools

import jax
import jax.numpy as jnp
from jax import lax
from jax.experimental import pallas as pl
from jax.experimental.pallas import tpu as pltpu
from jax.experimental.pallas import tpu_sc as plsc

VOCAB = 1000000
EMBED = 64
SEQ = 200
BATCH = 1024

NUM_CORES = 2
NUM_SUBCORES = 16
E_PER_CORE = EMBED // NUM_CORES
SMAX = 13
LANES = 16
CHUNK = 128
NCH = BATCH // CHUNK
VMAIN = 999936


def _sc_detile(table_t, tail_flat):
    mesh = plsc.VectorSubcoreMesh(core_axis_name="c", subcore_axis_name="s")

    @functools.partial(
        pl.kernel,
        mesh=mesh,
        out_type=jax.ShapeDtypeStruct((EMBED * VOCAB,), jnp.float32),
        scratch_types=[
            pltpu.VMEM((47616,), jnp.float32),
            pltpu.VMEM((47616,), jnp.float32),
            pltpu.SemaphoreType.DMA,
            pltpu.SemaphoreType.DMA,
        ],
        compiler_params=pltpu.CompilerParams(use_tc_tiling_on_sc=True,
                                             needs_layout_passes=False),
    )
    def k(tab_hbm, tail_hbm, out_hbm, buf0, buf1, sem_r, sem_w):
        chunks = [(i * 47616, 47616) for i in range(21)]
        _detile_body(tab_hbm, tail_hbm, out_hbm, (buf0, buf1), sem_r,
                     sem_w, chunks)

    return k(table_t, tail_flat)


def _detile_body(tab_hbm, tail_hbm, out_hbm, buf, sem_r, sem_w, chunks):
    c = lax.axis_index("c")
    t = lax.axis_index("s")
    wid = t * NUM_CORES + c
    e0 = wid * 2
    seq = [(e0 + r, off, ln) for r in range(2) for (off, ln) in chunks]
    n = len(seq)
    for r in range(2):
        pltpu.sync_copy(tail_hbm.at[pl.ds((e0 + r) * 64, 64)],
                        buf[0].at[pl.ds(r * 64, 64)])
    for r in range(2):
        pltpu.async_copy(buf[0].at[pl.ds(r * 64, 64)],
                         out_hbm.at[pl.ds((e0 + r) * VOCAB + VMAIN, 64)],
                         sem_w)
    for r in range(2):
        pltpu.make_async_copy(buf[0].at[pl.ds(r * 64, 64)],
                              out_hbm.at[pl.ds((e0 + r) * VOCAB + VMAIN, 64)],
                              sem_w).wait()
    e, off, ln = seq[0]
    pltpu.async_copy(tab_hbm.at[e, pl.ds(off, ln)], buf[0].at[pl.ds(0, ln)],
                     sem_r)
    for i, (e, off, ln) in enumerate(seq):
        b = buf[i % 2]
        pltpu.make_async_copy(tab_hbm.at[e, pl.ds(off, ln)],
                              b.at[pl.ds(0, ln)], sem_r).wait()
        if i >= 1:
            pe, poff, pln = seq[i - 1]
            pltpu.make_async_copy(tab_hbm.at[pe, pl.ds(poff, pln)],
                                  buf[1 - i % 2].at[pl.ds(0, pln)],
                                  sem_w).wait()
        if i + 1 < n:
            ne, noff, nln = seq[i + 1]
            pltpu.async_copy(tab_hbm.at[ne, pl.ds(noff, nln)],
                             buf[1 - i % 2].at[pl.ds(0, nln)], sem_r)
        pltpu.async_copy(b.at[pl.ds(0, ln)],
                         out_hbm.at[pl.ds(e * VOCAB + off, ln)], sem_w)
    e, off, ln = seq[n - 1]
    pltpu.make_async_copy(tab_hbm.at[e, pl.ds(off, ln)],
                          buf[(n - 1) % 2].at[pl.ds(0, ln)], sem_w).wait()


def _sc_embed(idx_flat, table_lin, pos2d):
    mesh = plsc.VectorSubcoreMesh(core_axis_name="c", subcore_axis_name="s")

    @functools.partial(
        pl.kernel,
        mesh=mesh,
        out_type=jax.ShapeDtypeStruct((SEQ, 8, NCH, 8, CHUNK), jnp.float32),
        scratch_types=[
            pltpu.VMEM_SHARED((2, VOCAB), jnp.float32),
            pltpu.VMEM((2, BATCH), jnp.int32),
            pltpu.VMEM((SMAX, EMBED), jnp.float32),
            pltpu.VMEM((2, NCH, CHUNK), jnp.float32),
            pltpu.SemaphoreType.DMA,
            pltpu.SemaphoreType.DMA,
            pltpu.SemaphoreType.DMA,
            pltpu.SemaphoreType.DMA,
        ],
        compiler_params=pltpu.CompilerParams(use_tc_tiling_on_sc=False,
                                             needs_layout_passes=False),
    )
    def k(idx_hbm, tab_hbm, pos_hbm, out_hbm, sp, idx_v, pos_v,
          dst_v, sem_st, sem_ix, sem_g, sem_o):
        c = lax.axis_index("c")
        t = lax.axis_index("s")
        s0 = (t * 25) // 2
        s1 = ((t + 1) * 25) // 2
        ns = s1 - s0
        e_base = c * E_PER_CORE

        pltpu.sync_copy(pos_hbm.at[pl.ds(s0, SMAX)], pos_v)

        @pl.when(t == 0)
        def _():
            pltpu.async_copy(tab_hbm.at[pl.ds(e_base * VOCAB, VOCAB)],
                             sp.at[0], sem_st)

        def e_body(j, carry):
            buf = j % 2
            pltpu.async_copy(idx_hbm.at[pl.ds(s0 * BATCH, BATCH)],
                             idx_v.at[0], sem_ix)
            pltpu.async_copy(idx_hbm.at[pl.ds((s0 + 1) * BATCH, BATCH)],
                             idx_v.at[1], sem_ix)
            plsc.subcore_barrier()

            @pl.when(t == 0)
            def _():
                pltpu.make_async_copy(
                    tab_hbm.at[pl.ds(0, VOCAB)],
                    sp.at[buf], sem_st).wait()

                @pl.when(j + 1 < E_PER_CORE)
                def _():
                    pltpu.async_copy(
                        tab_hbm.at[pl.ds((e_base + j + 1) * VOCAB, VOCAB)],
                        sp.at[(j + 1) % 2], sem_st)

            plsc.subcore_barrier()
            e = e_base + j
            tr = e // 8
            ei = e % 8

            pltpu.make_async_copy(idx_hbm.at[pl.ds(0, BATCH)], idx_v.at[0],
                                  sem_ix).wait()
            for cc in range(NCH):
                pltpu.async_copy(
                    sp.at[buf].at[idx_v.at[0, pl.ds(cc * CHUNK, CHUNK)]],
                    dst_v.at[0, cc], sem_g)

            def s_body(si, carry2):
                p = si % 2
                ps = jnp.zeros((LANES,), jnp.int32) + si
                pe = jnp.zeros((LANES,), jnp.int32) + e
                pos_splat = plsc.load_gather(pos_v, [ps, pe])
                for cc in range(NCH):
                    pltpu.make_async_copy(out_hbm.at[0, 0, 0, 0],
                                          dst_v.at[p, cc], sem_g).wait()
                    for u in range(CHUNK // LANES):
                        plsc.addupdate(
                            dst_v.at[p, cc, pl.ds(u * LANES, LANES)],
                            pos_splat,
                        )

                @pl.when(si + 2 < ns)
                def _():
                    pltpu.async_copy(
                        idx_hbm.at[pl.ds((s0 + si + 2) * BATCH, BATCH)],
                        idx_v.at[p], sem_ix)

                @pl.when(si >= 1)
                def _():
                    pltpu.make_async_copy(out_hbm.at[0, 0, :, 0],
                                          dst_v.at[1 - p], sem_o).wait()

                @pl.when(si + 1 < ns)
                def _():
                    pltpu.make_async_copy(idx_hbm.at[pl.ds(0, BATCH)],
                                          idx_v.at[1 - p], sem_ix).wait()

                    for cc in range(NCH):
                        pltpu.async_copy(
                            sp.at[buf].at[idx_v.at[1 - p,
                                                   pl.ds(cc * CHUNK, CHUNK)]],
                            dst_v.at[1 - p, cc], sem_g)

                pltpu.async_copy(dst_v.at[p], out_hbm.at[s0 + si, tr, :, ei],
                                 sem_o)
                return carry2

            lax.fori_loop(0, ns, s_body, 0)
            pltpu.make_async_copy(out_hbm.at[0, 0, :, 0],
                                  dst_v.at[(ns + 1) % 2], sem_o).wait()
            return carry

        lax.fori_loop(0, E_PER_CORE, e_body, 0)

    return k(idx_flat, table_lin, pos2d)


def kernel(idx, token_table, pos_table):
    idx_flat = idx.astype(jnp.int32).T.reshape(SEQ * BATCH)
    tail_flat = token_table[VMAIN:].T.reshape(64 * 64)
    table_lin = _sc_detile(token_table.T, tail_flat)
    out5 = _sc_embed(idx_flat, table_lin, pos_table)
    return out5.transpose(2, 4, 0, 1, 3).reshape(BATCH, SEQ, EMBED)
